# baseline (speedup 1.0000x reference)
import functools

import jax
import jax.numpy as jnp
from jax import lax
from jax.experimental import pallas as pl
from jax.experimental.pallas import tpu as pltpu
from jax.experimental.pallas import tpu_sc as plsc

N = 100000
D = 128
C = 128
NW = 32
N_FULL = N // C
TAIL = N - N_FULL * C
N_CHUNKS = N_FULL + 1
MAX_J = (N_CHUNKS + NW - 1) // NW


def _make_kernel():
  mesh = plsc.VectorSubcoreMesh(core_axis_name="c", subcore_axis_name="s")

  @functools.partial(
      pl.kernel,
      out_type=jax.ShapeDtypeStruct((N, D), jnp.float32),
      mesh=mesh,
      scratch_types=[
          pltpu.VMEM((C,), jnp.int32),
          pltpu.VMEM((C, D), jnp.float32),
          pltpu.VMEM((TAIL,), jnp.int32),
          pltpu.VMEM((TAIL, D), jnp.float32),
          pltpu.SemaphoreType.DMA,
      ],
  )
  def k(x_hbm, table_hbm, out_hbm, idx_v, rows_v, idx_t, rows_t, sem):
    wid = lax.axis_index("s") * 2 + lax.axis_index("c")

    def body(j, carry):
      cid = j * NW + wid

      @pl.when(cid < N_FULL)
      def _():
        base = cid * C
        pltpu.sync_copy(x_hbm.at[pl.ds(base, C)], idx_v)
        pltpu.async_copy(table_hbm.at[idx_v], rows_v, sem).wait()
        pltpu.sync_copy(rows_v, out_hbm.at[pl.ds(base, C)])

      @pl.when(cid == N_FULL)
      def _():
        base = N_FULL * C
        pltpu.sync_copy(x_hbm.at[pl.ds(base, TAIL)], idx_t)
        pltpu.async_copy(table_hbm.at[idx_t], rows_t, sem).wait()
        pltpu.sync_copy(rows_t, out_hbm.at[pl.ds(base, TAIL)])

      return carry

    lax.fori_loop(0, MAX_J, body, 0)

  return k


_lookup = _make_kernel()


def kernel(x, table):
  return _lookup(x.astype(jnp.int32), table)


# speedup vs baseline: 1.0183x; 1.0183x over previous
import functools

import jax
import jax.numpy as jnp
from jax import lax
from jax.experimental import pallas as pl
from jax.experimental.pallas import tpu as pltpu
from jax.experimental.pallas import tpu_sc as plsc

N = 100000
D = 128
C = 128
NW = 32
N_FULL = N // C
TAIL = N - N_FULL * C
CPW = (N_FULL + NW - 1) // NW
LAST_CH = N_FULL - (NW - 1) * CPW
NB = 4
STEPS = CPW + 2
OUTER = (STEPS + NB - 1) // NB


def _make_kernel():
  mesh = plsc.VectorSubcoreMesh(core_axis_name="c", subcore_axis_name="s")

  @functools.partial(
      pl.kernel,
      out_type=jax.ShapeDtypeStruct((N, D), jnp.float32),
      mesh=mesh,
      scratch_types=[
          pltpu.VMEM((CPW, C), jnp.int32),
          pltpu.VMEM((NB, C, D), jnp.float32),
          pltpu.VMEM((TAIL,), jnp.int32),
          pltpu.VMEM((TAIL, D), jnp.float32),
          pltpu.SemaphoreType.DMA((NB,)),
          pltpu.SemaphoreType.DMA((NB,)),
          pltpu.SemaphoreType.DMA,
      ],
  )
  def k(x3_hbm, x_hbm, table_hbm, out_hbm,
        idx_v, rows, idx_t, rows_t, sem_g, sem_w, sem_t):
    wid = lax.axis_index("s") * 2 + lax.axis_index("c")
    base_ch = wid * CPW
    n_my = jnp.where(wid == NW - 1, LAST_CH, CPW)

    pltpu.sync_copy(x3_hbm.at[wid], idx_v)

    def start_gather(j, b):
      pltpu.async_copy(table_hbm.at[idx_v.at[j]], rows.at[b], sem_g.at[b])

    def start_write(j, b):
      pltpu.async_copy(rows.at[b], out_hbm.at[pl.ds((base_ch + j) * C, C)],
                       sem_w.at[b])

    def wait_gather(b):
      pltpu.make_async_copy(out_hbm.at[pl.ds(0, C)], rows.at[b],
                            sem_g.at[b]).wait()

    def wait_write(b):
      pltpu.make_async_copy(rows.at[b], out_hbm.at[pl.ds(0, C)],
                            sem_w.at[b]).wait()

    for b in range(2):
      @pl.when(b < n_my)
      def _(b=b):
        start_gather(jnp.int32(b), b)

    def outer_body(jj, carry):
      for b in range(NB):
        j = jj * NB + b
        b2 = (b + 2) % NB

        @pl.when((j >= 2) & (j < n_my + 2))
        def _(b2=b2):
          wait_write(b2)

        @pl.when(j + 2 < n_my)
        def _(j=j, b2=b2):
          start_gather(j + 2, b2)

        @pl.when(j < n_my)
        def _(j=j, b=b):
          wait_gather(b)
          start_write(j, b)
      return carry

    lax.fori_loop(0, OUTER, outer_body, 0)

    @pl.when(wid == NW - 1)
    def _():
      pltpu.sync_copy(x_hbm.at[pl.ds(N_FULL * C, TAIL)], idx_t)
      pltpu.async_copy(table_hbm.at[idx_t], rows_t, sem_t).wait()
      pltpu.sync_copy(rows_t, out_hbm.at[pl.ds(N_FULL * C, TAIL)])

  return k


_lookup = _make_kernel()


def kernel(x, table):
  xi = x.astype(jnp.int32)
  x3 = jnp.pad(xi, (0, NW * CPW * C - N)).reshape(NW, CPW, C)
  return _lookup(x3, xi, table)


# speedup vs baseline: 7.1309x; 7.0026x over previous
import functools

import jax
import jax.numpy as jnp
from jax import lax
from jax.experimental import pallas as pl
from jax.experimental.pallas import tpu as pltpu
from jax.experimental.pallas import tpu_sc as plsc

N = 100000
D = 128
C = 128
NW = 32
N_FULL = N // C
TAIL = N - N_FULL * C
CPW = (N_FULL + NW - 1) // NW
LAST_CH = N_FULL - (NW - 1) * CPW
NB = 4
STEPS = CPW + 2
OUTER = (STEPS + NB - 1) // NB


def _make_kernel():
  mesh = plsc.VectorSubcoreMesh(core_axis_name="c", subcore_axis_name="s")

  @functools.partial(
      pl.kernel,
      out_type=jax.ShapeDtypeStruct((N, D), jnp.float32),
      mesh=mesh,
      scratch_types=[
          pltpu.VMEM((CPW, C), jnp.int32),
          pltpu.VMEM((NB, C, D), jnp.float32),
          pltpu.VMEM((TAIL,), jnp.int32),
          pltpu.VMEM((TAIL, D), jnp.float32),
          pltpu.SemaphoreType.DMA((NB,)),
          pltpu.SemaphoreType.DMA((NB,)),
          pltpu.SemaphoreType.DMA,
          pltpu.VMEM_SHARED((21, D), jnp.float32),
      ],
  )
  def k(x3_hbm, x_hbm, table_hbm, out_hbm,
        idx_v, rows, idx_t, rows_t, sem_g, sem_w, sem_t, table_s):
    wid = lax.axis_index("s") * 2 + lax.axis_index("c")
    base_ch = wid * CPW
    n_my = jnp.where(wid == NW - 1, LAST_CH, CPW)

    @pl.when(lax.axis_index("s") == 0)
    def _():
      pltpu.sync_copy(table_hbm, table_s)

    pltpu.sync_copy(x3_hbm.at[wid], idx_v)
    plsc.subcore_barrier()

    def start_gather(j, b):
      pltpu.async_copy(table_s.at[idx_v.at[j]], rows.at[b], sem_g.at[b])

    def start_write(j, b):
      pltpu.async_copy(rows.at[b], out_hbm.at[pl.ds((base_ch + j) * C, C)],
                       sem_w.at[b])

    def wait_gather(b):
      pltpu.make_async_copy(out_hbm.at[pl.ds(0, C)], rows.at[b],
                            sem_g.at[b]).wait()

    def wait_write(b):
      pltpu.make_async_copy(rows.at[b], out_hbm.at[pl.ds(0, C)],
                            sem_w.at[b]).wait()

    for b in range(2):
      @pl.when(b < n_my)
      def _(b=b):
        start_gather(jnp.int32(b), b)

    def outer_body(jj, carry):
      for b in range(NB):
        j = jj * NB + b
        b2 = (b + 2) % NB

        @pl.when((j >= 2) & (j < n_my + 2))
        def _(b2=b2):
          wait_write(b2)

        @pl.when(j + 2 < n_my)
        def _(j=j, b2=b2):
          start_gather(j + 2, b2)

        @pl.when(j < n_my)
        def _(j=j, b=b):
          wait_gather(b)
          start_write(j, b)
      return carry

    lax.fori_loop(0, OUTER, outer_body, 0)

    @pl.when(wid == NW - 1)
    def _():
      pltpu.sync_copy(x_hbm.at[pl.ds(N_FULL * C, TAIL)], idx_t)
      pltpu.async_copy(table_s.at[idx_t], rows_t, sem_t).wait()
      pltpu.sync_copy(rows_t, out_hbm.at[pl.ds(N_FULL * C, TAIL)])

  return k


_lookup = _make_kernel()


def kernel(x, table):
  xi = x.astype(jnp.int32)
  x3 = jnp.pad(xi, (0, NW * CPW * C - N)).reshape(NW, CPW, C)
  return _lookup(x3, xi, table)


# speedup vs baseline: 7.1393x; 1.0012x over previous
import functools

import jax
import jax.numpy as jnp
from jax import lax
from jax.experimental import pallas as pl
from jax.experimental.pallas import tpu as pltpu
from jax.experimental.pallas import tpu_sc as plsc

N = 100000
D = 128
V = 21
C = 128
NW = 32
N_FULL = N // C
TAIL = N - N_FULL * C
CPW = (N_FULL + NW - 1) // NW
LAST_CH = N_FULL - (NW - 1) * CPW
BPW = CPW * C
LAST_IDS = LAST_CH * C + TAIL
NB = 4
STEPS = CPW + 2
OUTER = (STEPS + NB - 1) // NB


def _make_kernel():
  mesh = plsc.VectorSubcoreMesh(core_axis_name="c", subcore_axis_name="s")

  @functools.partial(
      pl.kernel,
      out_type=jax.ShapeDtypeStruct((N, D), jnp.float32),
      mesh=mesh,
      scratch_types=[
          pltpu.VMEM((BPW,), jnp.int32),
          pltpu.VMEM((NB, C, D), jnp.float32),
          pltpu.VMEM((TAIL, D), jnp.float32),
          pltpu.SemaphoreType.DMA((NB,)),
          pltpu.SemaphoreType.DMA((NB,)),
          pltpu.SemaphoreType.DMA,
          pltpu.VMEM_SHARED((V, D), jnp.float32),
      ],
  )
  def k(x_hbm, table_hbm, out_hbm,
        idx_v, rows, rows_t, sem_g, sem_w, sem_t, table_s):
    wid = lax.axis_index("s") * 2 + lax.axis_index("c")
    base_ch = wid * CPW
    n_my = jnp.where(wid == NW - 1, LAST_CH, CPW)

    @pl.when(lax.axis_index("s") == 0)
    def _():
      pltpu.sync_copy(table_hbm, table_s)

    @pl.when(wid < NW - 1)
    def _():
      pltpu.sync_copy(x_hbm.at[pl.ds(wid * BPW, BPW)], idx_v)

    @pl.when(wid == NW - 1)
    def _():
      pltpu.sync_copy(x_hbm.at[pl.ds((NW - 1) * BPW, LAST_IDS)],
                      idx_v.at[pl.ds(0, LAST_IDS)])

    plsc.subcore_barrier()

    def start_gather(j, b):
      pltpu.async_copy(table_s.at[idx_v.at[pl.ds(j * C, C)]], rows.at[b],
                       sem_g.at[b])

    def start_write(j, b):
      pltpu.async_copy(rows.at[b], out_hbm.at[pl.ds((base_ch + j) * C, C)],
                       sem_w.at[b])

    def wait_gather(b):
      pltpu.make_async_copy(out_hbm.at[pl.ds(0, C)], rows.at[b],
                            sem_g.at[b]).wait()

    def wait_write(b):
      pltpu.make_async_copy(rows.at[b], out_hbm.at[pl.ds(0, C)],
                            sem_w.at[b]).wait()

    for b in range(2):
      @pl.when(b < n_my)
      def _(b=b):
        start_gather(jnp.int32(b), b)

    def outer_body(jj, carry):
      for b in range(NB):
        j = jj * NB + b
        b2 = (b + 2) % NB

        @pl.when((j >= 2) & (j < n_my + 2))
        def _(b2=b2):
          wait_write(b2)

        @pl.when(j + 2 < n_my)
        def _(j=j, b2=b2):
          start_gather(j + 2, b2)

        @pl.when(j < n_my)
        def _(j=j, b=b):
          wait_gather(b)
          start_write(j, b)
      return carry

    lax.fori_loop(0, OUTER, outer_body, 0)

    @pl.when(wid == NW - 1)
    def _():
      pltpu.async_copy(table_s.at[idx_v.at[pl.ds(LAST_CH * C, TAIL)]],
                       rows_t, sem_t).wait()
      pltpu.sync_copy(rows_t, out_hbm.at[pl.ds(N_FULL * C, TAIL)])

  return k


_lookup = _make_kernel()


def kernel(x, table):
  return _lookup(x.astype(jnp.int32), table)
